# baseline (speedup 1.0000x reference)
import functools

import jax
import jax.numpy as jnp
from jax import lax
from jax.experimental import pallas as pl
from jax.experimental.pallas import tpu as pltpu
from jax.experimental.pallas import tpu_sc as plsc

EPS = 1e-7

N_ANCHORS = 36864
NC, NS, L = 2, 16, 16
NW = NC * NS
APW = N_ANCHORS // NW
CPW = APW * 4


def _sc_regression_body(scores_hbm, ob_hbm, tb_hbm, out_hbm,
                        sc_v, ob_v, tb_v, ps_v, res_v,
                        sem_s, sem_ob, sem_tb):
    wid = lax.axis_index("s") * NC + lax.axis_index("c")
    base_a = wid * APW
    base_c = wid * CPW

    cp_s = pltpu.async_copy(scores_hbm.at[pl.ds(base_a, APW)], sc_v, sem_s)
    cp_ob = pltpu.async_copy(ob_hbm.at[pl.ds(base_c, CPW)], ob_v, sem_ob)
    cp_tb = pltpu.async_copy(tb_hbm.at[pl.ds(base_c, CPW)], tb_v, sem_tb)

    zeros = jnp.zeros((L,), jnp.float32)

    cp_s.wait()

    def score_body(i, carry):
        accp, accv = carry
        s = sc_v[pl.ds(i * L, L)]
        valid = jnp.where(s != -1.0, 1.0, 0.0)
        ps = jnp.where(s > 0.0, valid, 0.0)
        ps_v[pl.ds(i * L, L)] = ps
        return (accp + ps, accv + valid)

    accp, accv = lax.fori_loop(0, APW // L, score_body, (zeros, zeros))

    cp_ob.wait()
    cp_tb.wait()

    lane4 = lax.iota(jnp.int32, L) * 4

    def box_body(i, acca):
        off = 128 * (i // 32) + 16 * (i % 8)
        c = (i // 8) % 4
        o = plsc.load_gather(ob_v, [off * 4 + c + lane4])
        d = jnp.abs(tb_v[pl.ds(i * L, L)] - o)
        sl1 = jnp.where(d < 1.0, 0.5 * d * d, d - 0.5)
        w = ps_v[pl.ds(off, L)]
        return acca + w * sl1

    acca = lax.fori_loop(0, CPW // L, box_body, zeros)

    res_v[pl.ds(0, L)] = acca
    res_v[pl.ds(L, L)] = accp
    res_v[pl.ds(2 * L, L)] = accv
    pltpu.sync_copy(res_v, out_hbm.at[wid])


@functools.lru_cache(maxsize=1)
def _sc_regression():
    return pl.kernel(
        _sc_regression_body,
        compiler_params=pltpu.CompilerParams(needs_layout_passes=False),
        out_type=jax.ShapeDtypeStruct((NW, 3 * L), jnp.float32),
        mesh=plsc.VectorSubcoreMesh(core_axis_name="c", subcore_axis_name="s",
                                    num_cores=NC, num_subcores=NS),
        scratch_types=[
            pltpu.VMEM((APW,), jnp.float32),
            pltpu.VMEM((CPW,), jnp.float32),
            pltpu.VMEM((CPW,), jnp.float32),
            pltpu.VMEM((APW,), jnp.float32),
            pltpu.VMEM((3 * L,), jnp.float32),
            pltpu.SemaphoreType.DMA,
            pltpu.SemaphoreType.DMA,
            pltpu.SemaphoreType.DMA,
        ],
    )


def _tc_bce_body(ts_ref, os_ref, sum_ref, cnt_ref):
    t = ts_ref[...]
    p = jnp.clip(os_ref[...], EPS, 1.0 - EPS)
    bce = -(t * jnp.log(p) + (1.0 - t) * jnp.log(1.0 - p))
    mask = (t != -1.0).astype(jnp.float32)
    sum_ref[0, 0] = jnp.sum(bce * mask)
    cnt_ref[0, 0] = jnp.sum(mask)


def _tc_bce(target_scores_2d, output_scores_2d):
    return pl.pallas_call(
        _tc_bce_body,
        out_shape=(
            jax.ShapeDtypeStruct((1, 1), jnp.float32),
            jax.ShapeDtypeStruct((1, 1), jnp.float32),
        ),
        out_specs=(
            pl.BlockSpec(memory_space=pltpu.SMEM),
            pl.BlockSpec(memory_space=pltpu.SMEM),
        ),
    )(target_scores_2d, output_scores_2d)


def _planar(boxes):
    return boxes.reshape(288, 128, 4).transpose(0, 2, 1).reshape(-1)


def kernel(output_bounding_boxes, target_bounding_boxes, output_scores, target_scores):
    scores = output_scores.reshape(-1)
    ob = output_bounding_boxes.reshape(-1)
    tb = _planar(target_bounding_boxes)

    partials = _sc_regression()(scores, ob, tb)
    scores_lin = lax.optimization_barrier(scores)
    bce_sum, cls_cnt = _tc_bce(target_scores.reshape(288, 128),
                               scores_lin.reshape(288, 128))
    sums = jnp.sum(partials.reshape(NW, 3, L), axis=(0, 2))
    classification_loss = bce_sum[0, 0] / cls_cnt[0, 0]
    regression_loss = 10.0 * (sums[0] / (sums[1] + sums[2] * EPS))
    return classification_loss + regression_loss


# speedup vs baseline: 1.0153x; 1.0153x over previous
import functools

import jax
import jax.numpy as jnp
from jax import lax
from jax.experimental import pallas as pl
from jax.experimental.pallas import tpu as pltpu
from jax.experimental.pallas import tpu_sc as plsc

EPS = 1e-7

N_ANCHORS = 36864
NC, NS, L = 2, 16, 16
NW = NC * NS
APW = N_ANCHORS // NW
CPW = APW * 4


def _sc_regression_body(scores_hbm, ob_hbm, tb_hbm, out_hbm,
                        sc_v, ob_v, tb_v, res_v,
                        sem_s, sem_ob, sem_tb):
    wid = lax.axis_index("s") * NC + lax.axis_index("c")
    base_a = wid * APW
    base_c = wid * CPW

    cp_s = pltpu.async_copy(scores_hbm.at[pl.ds(base_a, APW)], sc_v, sem_s)
    cp_ob = pltpu.async_copy(ob_hbm.at[pl.ds(base_c, CPW)], ob_v, sem_ob)
    cp_tb = pltpu.async_copy(tb_hbm.at[pl.ds(base_c, CPW)], tb_v, sem_tb)

    zeros = jnp.zeros((L,), jnp.float32)
    lane4 = lax.iota(jnp.int32, L) * 4

    cp_s.wait()
    cp_ob.wait()
    cp_tb.wait()

    def block_body(b, carry):
        acc0, acc1, acc2, acc3, accp, accv = carry
        accs = [acc0, acc1, acc2, acc3]
        a128 = b * L * 8
        ws = []
        for m in range(8):
            s = sc_v[pl.ds(a128 + m * L, L)]
            valid = jnp.where(s != -1.0, 1.0, 0.0)
            ps = jnp.where(s > 0.0, valid, 0.0)
            accp = accp + ps
            accv = accv + valid
            ws.append(ps)
        for c in range(4):
            acc = accs[c]
            for m in range(8):
                f = b * 512 + c * 128 + m * L
                o = plsc.load_gather(ob_v, [(a128 + m * L) * 4 + c + lane4])
                d = jnp.abs(tb_v[pl.ds(f, L)] - o)
                sl1 = jnp.where(d < 1.0, 0.5 * d * d, d - 0.5)
                acc = acc + ws[m] * sl1
            accs[c] = acc
        return (accs[0], accs[1], accs[2], accs[3], accp, accv)

    acc0, acc1, acc2, acc3, accp, accv = lax.fori_loop(
        0, APW // (L * 8), block_body,
        (zeros, zeros, zeros, zeros, zeros, zeros))

    res_v[pl.ds(0, L)] = (acc0 + acc1) + (acc2 + acc3)
    res_v[pl.ds(L, L)] = accp
    res_v[pl.ds(2 * L, L)] = accv
    pltpu.sync_copy(res_v, out_hbm.at[wid])


@functools.lru_cache(maxsize=1)
def _sc_regression():
    return pl.kernel(
        _sc_regression_body,
        compiler_params=pltpu.CompilerParams(needs_layout_passes=False),
        out_type=jax.ShapeDtypeStruct((NW, 3 * L), jnp.float32),
        mesh=plsc.VectorSubcoreMesh(core_axis_name="c", subcore_axis_name="s",
                                    num_cores=NC, num_subcores=NS),
        scratch_types=[
            pltpu.VMEM((APW,), jnp.float32),
            pltpu.VMEM((CPW,), jnp.float32),
            pltpu.VMEM((CPW,), jnp.float32),
            pltpu.VMEM((3 * L,), jnp.float32),
            pltpu.SemaphoreType.DMA,
            pltpu.SemaphoreType.DMA,
            pltpu.SemaphoreType.DMA,
        ],
    )


def _tc_bce_body(ts_ref, os_ref, part_ref, out_ref):
    t = ts_ref[...]
    p = jnp.clip(os_ref[...], EPS, 1.0 - EPS)
    bce = -(t * jnp.log(p) + (1.0 - t) * jnp.log(1.0 - p))
    mask = (t != -1.0).astype(jnp.float32)
    classification_loss = jnp.sum(bce * mask) / jnp.sum(mask)
    parts = part_ref[...].reshape(NW, 3, L)
    a = jnp.sum(parts[:, 0, :])
    bp = jnp.sum(parts[:, 1, :])
    vm = jnp.sum(parts[:, 2, :])
    regression_loss = 10.0 * (a / (bp + vm * EPS))
    out_ref[0, 0] = classification_loss + regression_loss


def _tc_bce(target_scores_2d, output_scores_2d, partials):
    return pl.pallas_call(
        _tc_bce_body,
        out_shape=jax.ShapeDtypeStruct((1, 1), jnp.float32),
        out_specs=pl.BlockSpec(memory_space=pltpu.SMEM),
    )(target_scores_2d, output_scores_2d, partials)


def _planar(boxes):
    return boxes.reshape(288, 128, 4).transpose(0, 2, 1).reshape(-1)


def kernel(output_bounding_boxes, target_bounding_boxes, output_scores, target_scores):
    scores = output_scores.reshape(-1)
    ob = output_bounding_boxes.reshape(-1)
    tb = _planar(target_bounding_boxes)

    partials = _sc_regression()(scores, ob, tb)
    scores_lin = lax.optimization_barrier(scores)
    loss = _tc_bce(target_scores.reshape(288, 128),
                   scores_lin.reshape(288, 128), partials)
    return loss.reshape(())


# speedup vs baseline: 1.0914x; 1.0750x over previous
import functools

import jax
import jax.numpy as jnp
from jax import lax
from jax.experimental import pallas as pl
from jax.experimental.pallas import tpu as pltpu
from jax.experimental.pallas import tpu_sc as plsc

EPS = 1e-7

N_ANCHORS = 36864
NC, NS, L = 2, 16, 16
NW = NC * NS
APW = N_ANCHORS // NW
CPW = APW * 4


def _sc_regression_body(scores_hbm, ob_hbm, tb_hbm, out_hbm,
                        sc_v, ob_v, tb_v, ps_v, res_v,
                        sem_s, sem_ob, sem_tb):
    wid = lax.axis_index("s") * NC + lax.axis_index("c")
    base_a = wid * APW
    base_c = wid * CPW

    cp_s = pltpu.async_copy(scores_hbm.at[pl.ds(base_a, APW)], sc_v, sem_s)
    cp_ob = pltpu.async_copy(ob_hbm.at[pl.ds(base_c, CPW)], ob_v, sem_ob)
    cp_tb = pltpu.async_copy(tb_hbm.at[pl.ds(base_c, CPW)], tb_v, sem_tb)

    zeros = jnp.zeros((L,), jnp.float32)
    lane4 = lax.iota(jnp.int32, L) * 4

    cp_s.wait()

    @plsc.parallel_loop(0, APW // L, unroll=4, carry=(zeros, zeros))
    def score_loop(i, carry):
        accp, accv = carry
        s = sc_v[pl.ds(i * L, L)]
        valid = jnp.where(s != -1.0, 1.0, 0.0)
        ps = jnp.where(s > 0.0, valid, 0.0)
        ps_v[pl.ds(i * L, L)] = ps
        return (accp + ps, accv + valid)

    accp, accv = score_loop

    cp_ob.wait()
    cp_tb.wait()

    @plsc.parallel_loop(0, CPW // L, unroll=4, carry=zeros)
    def box_loop(i, acca):
        off = 128 * (i // 32) + 16 * (i % 8)
        c = (i // 8) % 4
        o = plsc.load_gather(ob_v, [off * 4 + c + lane4])
        d = jnp.abs(tb_v[pl.ds(i * L, L)] - o)
        sl1 = jnp.where(d < 1.0, 0.5 * d * d, d - 0.5)
        w = ps_v[pl.ds(off, L)]
        return acca + w * sl1

    acca = box_loop

    res_v[pl.ds(0, L)] = acca
    res_v[pl.ds(L, L)] = accp
    res_v[pl.ds(2 * L, L)] = accv
    pltpu.sync_copy(res_v, out_hbm.at[wid])


@functools.lru_cache(maxsize=1)
def _sc_regression():
    return pl.kernel(
        _sc_regression_body,
        compiler_params=pltpu.CompilerParams(needs_layout_passes=False),
        out_type=jax.ShapeDtypeStruct((NW, 3 * L), jnp.float32),
        mesh=plsc.VectorSubcoreMesh(core_axis_name="c", subcore_axis_name="s",
                                    num_cores=NC, num_subcores=NS),
        scratch_types=[
            pltpu.VMEM((APW,), jnp.float32),
            pltpu.VMEM((CPW,), jnp.float32),
            pltpu.VMEM((CPW,), jnp.float32),
            pltpu.VMEM((APW,), jnp.float32),
            pltpu.VMEM((3 * L,), jnp.float32),
            pltpu.SemaphoreType.DMA,
            pltpu.SemaphoreType.DMA,
            pltpu.SemaphoreType.DMA,
        ],
    )


def _tc_bce_body(ts_ref, os_ref, part_ref, out_ref):
    t = ts_ref[...]
    p = jnp.clip(os_ref[...], EPS, 1.0 - EPS)
    bce = -(t * jnp.log(p) + (1.0 - t) * jnp.log(1.0 - p))
    mask = (t != -1.0).astype(jnp.float32)
    classification_loss = jnp.sum(bce * mask) / jnp.sum(mask)
    parts = part_ref[...].reshape(NW, 3, L)
    a = jnp.sum(parts[:, 0, :])
    bp = jnp.sum(parts[:, 1, :])
    vm = jnp.sum(parts[:, 2, :])
    regression_loss = 10.0 * (a / (bp + vm * EPS))
    out_ref[0, 0] = classification_loss + regression_loss


def _tc_bce(target_scores_2d, output_scores_2d, partials):
    return pl.pallas_call(
        _tc_bce_body,
        out_shape=jax.ShapeDtypeStruct((1, 1), jnp.float32),
        out_specs=pl.BlockSpec(memory_space=pltpu.SMEM),
    )(target_scores_2d, output_scores_2d, partials)


def _planar(boxes):
    return boxes.reshape(288, 128, 4).transpose(0, 2, 1).reshape(-1)


def kernel(output_bounding_boxes, target_bounding_boxes, output_scores, target_scores):
    scores = output_scores.reshape(-1)
    ob = output_bounding_boxes.reshape(-1)
    tb = _planar(target_bounding_boxes)

    partials = _sc_regression()(scores, ob, tb)
    scores_lin = lax.optimization_barrier(scores)
    loss = _tc_bce(target_scores.reshape(288, 128),
                   scores_lin.reshape(288, 128), partials)
    return loss.reshape(())
